# baseline (speedup 1.0000x reference)
import functools

import jax
import jax.numpy as jnp
from jax import lax
from jax.experimental import pallas as pl
from jax.experimental.pallas import tpu as pltpu
from jax.experimental.pallas import tpu_sc as plsc

N = 10000
E = 320000
D = 128
G = 64

NC = 2
NS = 16
NW = NC * NS
L = 16

CH = 128
EPW = 10240
PADW = EPW - E // NW
KJ = EPW // CH
NACC = 10240
ORT = 624
OCH = 48
OTAIL = N - NS * ORT
NG = 4
KJG = KJ // NG

NPAD = 10240
NPW = 320
PCH = 80

_mesh = plsc.VectorSubcoreMesh(core_axis_name="c", subcore_axis_name="s")


def _zero_1d(ref, n):
    @pl.loop(0, n, step=L)
    def _(i):
        ref[pl.ds(i, L)] = jnp.zeros((L,), jnp.float32)


def _fill_rows(ref, rows, val):
    @pl.loop(0, rows)
    def _(r):
        @pl.loop(0, ref.shape[1], step=L)
        def _(cc):
            ref[r, pl.ds(cc, L)] = jnp.full((L,), val, jnp.float32)


NPT = NPAD // NS


@functools.partial(
    pl.kernel,
    out_type=jax.ShapeDtypeStruct((NC * NPAD,), jnp.float32),
    mesh=_mesh,
    scratch_types=[
        pltpu.VMEM((KJ, CH), jnp.int32),
        pltpu.VMEM((CH,), jnp.float32),
        pltpu.VMEM((NPT,), jnp.float32),
        pltpu.VMEM_SHARED((NPAD,), jnp.float32),
    ],
)
def _deg_kernel(dst_hbm, out_hbm, idx_v, ones_v, buf_v, deg_sh):
    c = lax.axis_index("c")
    s = lax.axis_index("s")
    w = c * NS + s
    _zero_1d(buf_v, NPT)
    pltpu.sync_copy(buf_v, deg_sh.at[pl.ds(s * NPT, NPT)])

    @pl.loop(0, CH, step=L)
    def _(i):
        ones_v[pl.ds(i, L)] = jnp.ones((L,), jnp.float32)

    pltpu.sync_copy(dst_hbm.at[w], idx_v)
    plsc.subcore_barrier()

    @pl.loop(0, KJ)
    def _(j):
        pltpu.sync_copy(ones_v, deg_sh.at[idx_v.at[j]], add=True)

    plsc.subcore_barrier()
    pltpu.sync_copy(deg_sh.at[pl.ds(s * NPT, NPT)], buf_v)
    pltpu.sync_copy(buf_v, out_hbm.at[pl.ds(c * NPAD + s * NPT, NPT)])


@functools.partial(
    pl.kernel,
    out_type=jax.ShapeDtypeStruct((NC, N, D), jnp.float32),
    mesh=_mesh,
    scratch_types=[
        pltpu.VMEM((KJG, CH), jnp.int32),
        pltpu.VMEM((KJG, CH), jnp.int32),
        pltpu.VMEM((CH, D), jnp.float32),
        pltpu.VMEM((CH, D), jnp.float32),
        pltpu.VMEM((OCH, D), jnp.float32),
        pltpu.VMEM_SHARED((NACC, D), jnp.float32),
        pltpu.SemaphoreType.DMA,
        pltpu.SemaphoreType.DMA,
    ],
)
def _edge_kernel(g_hbm, src_hbm, dst_hbm, out_hbm, src_v, dst_v, r0_v, r1_v,
                 buf_v, acc_sh, sem0, sem1):
    c = lax.axis_index("c")
    s = lax.axis_index("s")
    w = c * NS + s
    _fill_rows(buf_v, OCH, 0.0)

    @pl.loop(0, ORT // OCH)
    def _(z):
        pltpu.sync_copy(buf_v, acc_sh.at[pl.ds(s * ORT + z * OCH, OCH)])

    @pl.when(s == NS - 1)
    def _():
        pltpu.sync_copy(buf_v.at[pl.ds(0, OTAIL)],
                        acc_sh.at[pl.ds(NS * ORT, OTAIL)])

    plsc.subcore_barrier()

    @pl.loop(0, NG)
    def _(gi):
        pltpu.sync_copy(src_hbm.at[w, gi], src_v)
        pltpu.sync_copy(dst_hbm.at[w, gi], dst_v)

        @pl.loop(0, KJG)
        def _(j):
            pltpu.sync_copy(g_hbm.at[src_v.at[j]], r0_v)
            pltpu.sync_copy(r0_v, acc_sh.at[dst_v.at[j]], add=True)

    plsc.subcore_barrier()

    @pl.loop(0, ORT // OCH)
    def _(z):
        r0 = s * ORT + z * OCH
        pltpu.sync_copy(acc_sh.at[pl.ds(r0, OCH)], buf_v)
        pltpu.sync_copy(buf_v, out_hbm.at[c, pl.ds(r0, OCH)])

    @pl.when(s == NS - 1)
    def _():
        pltpu.sync_copy(acc_sh.at[pl.ds(NS * ORT, OTAIL)],
                        buf_v.at[pl.ds(0, OTAIL)])
        pltpu.sync_copy(buf_v.at[pl.ds(0, OTAIL)],
                        out_hbm.at[c, pl.ds(NS * ORT, OTAIL)])


GPT = 8
CPAD = 128


@functools.partial(
    pl.kernel,
    out_type=(
        jax.ShapeDtypeStruct((NC, G, D), jnp.float32),
        jax.ShapeDtypeStruct((NC * CPAD,), jnp.float32),
    ),
    mesh=_mesh,
    scratch_types=[
        pltpu.VMEM((PCH,), jnp.int32),
        pltpu.VMEM((PCH,), jnp.float32),
        pltpu.VMEM((PCH, D), jnp.float32),
        pltpu.VMEM((GPT, D), jnp.float32),
        pltpu.VMEM((CPAD,), jnp.float32),
        pltpu.VMEM_SHARED((G, D), jnp.float32),
        pltpu.VMEM_SHARED((CPAD,), jnp.float32),
    ],
)
def _pool_kernel(h_hbm, batch_hbm, pool_hbm, cnt_hbm, bidx_v, ones_v, rows_v,
                 pbuf_v, cbuf_v, pool_sh, cnt_sh):
    c = lax.axis_index("c")
    s = lax.axis_index("s")
    w = c * NS + s
    _fill_rows(pbuf_v, GPT, 0.0)
    _zero_1d(cbuf_v, CPAD)

    @pl.when(s < G // GPT)
    def _():
        pltpu.sync_copy(pbuf_v, pool_sh.at[pl.ds(s * GPT, GPT)])

    @pl.when(s == 0)
    def _():
        pltpu.sync_copy(cbuf_v, cnt_sh)

    @pl.loop(0, PCH, step=L)
    def _(i):
        ones_v[pl.ds(i, L)] = jnp.ones((L,), jnp.float32)

    plsc.subcore_barrier()

    nchunks = jnp.where(w == NW - 1, 1, NPW // PCH)

    @pl.loop(0, nchunks)
    def _(k):
        base = w * NPW + k * PCH
        pltpu.sync_copy(batch_hbm.at[pl.ds(base, PCH)], bidx_v)
        pltpu.sync_copy(h_hbm.at[pl.ds(base, PCH)], rows_v)
        pltpu.sync_copy(rows_v, pool_sh.at[bidx_v], add=True)
        pltpu.sync_copy(ones_v, cnt_sh.at[bidx_v], add=True)

    plsc.subcore_barrier()

    @pl.when(s < G // GPT)
    def _():
        pltpu.sync_copy(pool_sh.at[pl.ds(s * GPT, GPT)], pbuf_v)
        pltpu.sync_copy(pbuf_v, pool_hbm.at[c, pl.ds(s * GPT, GPT)])

    @pl.when(s == 0)
    def _():
        pltpu.sync_copy(cnt_sh, cbuf_v)
        pltpu.sync_copy(cbuf_v, cnt_hbm.at[pl.ds(c * CPAD, CPAD)])


def _dinv_body(dp_ref, o_ref):
    deg = jnp.sum(dp_ref[...], axis=0, keepdims=True) + 1.0
    o_ref[...] = lax.rsqrt(deg)


def _mm_pre_body(x_ref, w_ref, dinv_ref, o_ref):
    h = jnp.dot(x_ref[...], w_ref[...], preferred_element_type=jnp.float32)
    o_ref[...] = dinv_ref[...] * h


def _combine_relu_body(s_ref, g_ref, dinv_ref, b_ref, o_ref):
    v = dinv_ref[...] * (s_ref[0] + s_ref[1] + g_ref[...]) + b_ref[...]
    o_ref[...] = jnp.maximum(v, 0.0)


def _combine_body(s_ref, g_ref, dinv_ref, b_ref, o_ref):
    o_ref[...] = dinv_ref[...] * (s_ref[0] + s_ref[1] + g_ref[...]) + b_ref[...]


def _final_body(pp_ref, cc_ref, wl_ref, bl_ref, o_ref):
    def body(i, carry):
        p, c2 = carry
        return p + pp_ref[i], c2 + cc_ref[i]

    p0 = jnp.zeros((G, D), jnp.float32)
    c0 = jnp.zeros((G, 1), jnp.float32)
    p, c2 = lax.fori_loop(0, NC, body, (p0, c0))
    pooled = p / jnp.maximum(c2, 1.0)
    o_ref[...] = (
        jnp.dot(pooled, wl_ref[...], preferred_element_type=jnp.float32)
        + bl_ref[...]
    )


BR = 1000


def _mm_pre(x, w, dinv):
    return pl.pallas_call(
        _mm_pre_body,
        grid=(N // BR,),
        in_specs=[
            pl.BlockSpec((BR, D), lambda i: (i, 0)),
            pl.BlockSpec((D, D), lambda i: (0, 0)),
            pl.BlockSpec((BR, 1), lambda i: (i, 0)),
        ],
        out_specs=pl.BlockSpec((BR, D), lambda i: (i, 0)),
        out_shape=jax.ShapeDtypeStruct((N, D), jnp.float32),
    )(x, w, dinv)


def _combine(srt, g, dinv, b, relu):
    body = _combine_relu_body if relu else _combine_body
    return pl.pallas_call(
        body,
        grid=(N // BR,),
        in_specs=[
            pl.BlockSpec((NC, BR, D), lambda i: (0, i, 0)),
            pl.BlockSpec((BR, D), lambda i: (i, 0)),
            pl.BlockSpec((BR, 1), lambda i: (i, 0)),
            pl.BlockSpec((1, D), lambda i: (0, 0)),
        ],
        out_specs=pl.BlockSpec((BR, D), lambda i: (i, 0)),
        out_shape=jax.ShapeDtypeStruct((N, D), jnp.float32),
    )(srt, g, dinv, b)


def kernel(x, edge_index, batch, W1, b1, W2, b2, W3, b3, Wl, bl):
    pad_src = jnp.zeros((NW, PADW), jnp.int32)
    pad_dst = jnp.broadcast_to(
        N + jnp.arange(PADW, dtype=jnp.int32), (NW, PADW))
    src = jnp.concatenate(
        [edge_index[0].reshape(NW, E // NW), pad_src], axis=1
    ).reshape(NW, NG, KJG, CH)
    dst = jnp.concatenate(
        [edge_index[1].reshape(NW, E // NW), pad_dst], axis=1
    ).reshape(NW, NG, KJG, CH)

    deg_part = _deg_kernel(dst.reshape(NW, KJ, CH)).reshape(NC, NPAD)
    dinv_flat = pl.pallas_call(
        _dinv_body,
        out_shape=jax.ShapeDtypeStruct((1, NPAD), jnp.float32),
    )(deg_part)
    dinv = dinv_flat.reshape(NPAD)[:N].reshape(N, 1)

    h = x
    for w, b, relu in ((W1, b1, True), (W2, b2, True), (W3, b3, False)):
        g = _mm_pre(h, w, dinv)
        srt = _edge_kernel(g, src, dst)
        h = _combine(srt, g, dinv, b.reshape(1, D), relu)

    pool_part, cnt_part = _pool_kernel(h, batch)
    cnt3 = cnt_part.reshape(NC, CPAD)[:, :G].reshape(NC, G, 1)
    y = pl.pallas_call(
        _final_body,
        out_shape=jax.ShapeDtypeStruct((G, 1), jnp.float32),
    )(pool_part, cnt3, Wl, bl.reshape(1, 1))
    return y


# speedup vs baseline: 2.4354x; 2.4354x over previous
import functools

import jax
import jax.numpy as jnp
from jax import lax
from jax.experimental import pallas as pl
from jax.experimental.pallas import tpu as pltpu
from jax.experimental.pallas import tpu_sc as plsc

N = 10000
E = 320000
D = 128
G = 64

NC = 2
NS = 16
NW = NC * NS
L = 16

CH = 80
EPW = E // NW
KJ = EPW // CH
NACC = N
ORT = 624
OCH = 48
OTAIL = N - NS * ORT
NG = 5
KJG = KJ // NG

NPAD = 10240
NPW = 320
PCH = 80

_mesh = plsc.VectorSubcoreMesh(core_axis_name="c", subcore_axis_name="s")


def _zero_1d(ref, n):
    @pl.loop(0, n, step=L)
    def _(i):
        ref[pl.ds(i, L)] = jnp.zeros((L,), jnp.float32)


def _fill_rows(ref, rows, val):
    @pl.loop(0, rows)
    def _(r):
        @pl.loop(0, ref.shape[1], step=L)
        def _(cc):
            ref[r, pl.ds(cc, L)] = jnp.full((L,), val, jnp.float32)


NPT = NPAD // NS


@functools.partial(
    pl.kernel,
    out_type=jax.ShapeDtypeStruct((NC * NPAD,), jnp.float32),
    mesh=_mesh,
    scratch_types=[
        pltpu.VMEM((KJ, CH), jnp.int32),
        pltpu.VMEM((CH,), jnp.float32),
        pltpu.VMEM((NPT,), jnp.float32),
        pltpu.VMEM_SHARED((NPAD,), jnp.float32),
    ],
)
def _deg_kernel(dst_hbm, out_hbm, idx_v, ones_v, buf_v, deg_sh):
    c = lax.axis_index("c")
    s = lax.axis_index("s")
    w = c * NS + s
    _zero_1d(buf_v, NPT)
    pltpu.sync_copy(buf_v, deg_sh.at[pl.ds(s * NPT, NPT)])

    @pl.loop(0, CH, step=L)
    def _(i):
        ones_v[pl.ds(i, L)] = jnp.ones((L,), jnp.float32)

    pltpu.sync_copy(dst_hbm.at[w], idx_v)
    plsc.subcore_barrier()

    @pl.loop(0, KJ)
    def _(j):
        pltpu.sync_copy(ones_v, deg_sh.at[idx_v.at[j]], add=True)

    plsc.subcore_barrier()
    pltpu.sync_copy(deg_sh.at[pl.ds(s * NPT, NPT)], buf_v)
    pltpu.sync_copy(buf_v, out_hbm.at[pl.ds(c * NPAD + s * NPT, NPT)])


@functools.partial(
    pl.kernel,
    out_type=jax.ShapeDtypeStruct((NC, N, D), jnp.float32),
    mesh=_mesh,
    scratch_types=[
        pltpu.VMEM((KJG, CH), jnp.int32),
        pltpu.VMEM((KJG, CH), jnp.int32),
        pltpu.VMEM((CH, D), jnp.float32),
        pltpu.VMEM((CH, D), jnp.float32),
        pltpu.VMEM((OCH, D), jnp.float32),
        pltpu.VMEM_SHARED((NACC, D), jnp.float32),
        pltpu.SemaphoreType.DMA,
        pltpu.SemaphoreType.DMA,
    ],
)
def _edge_kernel(g_hbm, src_hbm, dst_hbm, out_hbm, src_v, dst_v, r0_v, r1_v,
                 buf_v, acc_sh, sem0, sem1):
    c = lax.axis_index("c")
    s = lax.axis_index("s")
    w = c * NS + s
    _fill_rows(buf_v, OCH, 0.0)

    @pl.loop(0, ORT // OCH)
    def _(z):
        pltpu.sync_copy(buf_v, acc_sh.at[pl.ds(s * ORT + z * OCH, OCH)])

    @pl.when(s == NS - 1)
    def _():
        pltpu.sync_copy(buf_v.at[pl.ds(0, OTAIL)],
                        acc_sh.at[pl.ds(NS * ORT, OTAIL)])

    plsc.subcore_barrier()

    def _wait_sc(buf, sem):
        pltpu.make_async_copy(buf, acc_sh.at[dst_v.at[0]], sem).wait()

    @pl.loop(0, NG)
    def _(gi):
        pltpu.sync_copy(src_hbm.at[w, gi], src_v)
        pltpu.sync_copy(dst_hbm.at[w, gi], dst_v)
        pltpu.sync_copy(g_hbm.at[src_v.at[0]], r0_v)
        pltpu.async_copy(r0_v, acc_sh.at[dst_v.at[0]], sem0, add=True)
        pltpu.sync_copy(g_hbm.at[src_v.at[1]], r1_v)
        pltpu.async_copy(r1_v, acc_sh.at[dst_v.at[1]], sem1, add=True)

        @pl.loop(1, (KJG - 1) // 2)
        def _(k):
            j = 2 * k
            _wait_sc(r0_v, sem0)
            pltpu.sync_copy(g_hbm.at[src_v.at[j]], r0_v)
            pltpu.async_copy(r0_v, acc_sh.at[dst_v.at[j]], sem0, add=True)
            _wait_sc(r1_v, sem1)
            pltpu.sync_copy(g_hbm.at[src_v.at[j + 1]], r1_v)
            pltpu.async_copy(r1_v, acc_sh.at[dst_v.at[j + 1]], sem1, add=True)

        _wait_sc(r0_v, sem0)
        pltpu.sync_copy(g_hbm.at[src_v.at[KJG - 1]], r0_v)
        pltpu.async_copy(r0_v, acc_sh.at[dst_v.at[KJG - 1]], sem0, add=True)
        _wait_sc(r0_v, sem0)
        _wait_sc(r1_v, sem1)

    plsc.subcore_barrier()

    @pl.loop(0, ORT // OCH)
    def _(z):
        r0 = s * ORT + z * OCH
        pltpu.sync_copy(acc_sh.at[pl.ds(r0, OCH)], buf_v)
        pltpu.sync_copy(buf_v, out_hbm.at[c, pl.ds(r0, OCH)])

    @pl.when(s == NS - 1)
    def _():
        pltpu.sync_copy(acc_sh.at[pl.ds(NS * ORT, OTAIL)],
                        buf_v.at[pl.ds(0, OTAIL)])
        pltpu.sync_copy(buf_v.at[pl.ds(0, OTAIL)],
                        out_hbm.at[c, pl.ds(NS * ORT, OTAIL)])


GPT = 8
CPAD = 128


@functools.partial(
    pl.kernel,
    out_type=(
        jax.ShapeDtypeStruct((NC, G, D), jnp.float32),
        jax.ShapeDtypeStruct((NC * CPAD,), jnp.float32),
    ),
    mesh=_mesh,
    scratch_types=[
        pltpu.VMEM((PCH,), jnp.int32),
        pltpu.VMEM((PCH,), jnp.float32),
        pltpu.VMEM((PCH, D), jnp.float32),
        pltpu.VMEM((GPT, D), jnp.float32),
        pltpu.VMEM((CPAD,), jnp.float32),
        pltpu.VMEM_SHARED((G, D), jnp.float32),
        pltpu.VMEM_SHARED((CPAD,), jnp.float32),
    ],
)
def _pool_kernel(h_hbm, batch_hbm, pool_hbm, cnt_hbm, bidx_v, ones_v, rows_v,
                 pbuf_v, cbuf_v, pool_sh, cnt_sh):
    c = lax.axis_index("c")
    s = lax.axis_index("s")
    w = c * NS + s
    _fill_rows(pbuf_v, GPT, 0.0)
    _zero_1d(cbuf_v, CPAD)

    @pl.when(s < G // GPT)
    def _():
        pltpu.sync_copy(pbuf_v, pool_sh.at[pl.ds(s * GPT, GPT)])

    @pl.when(s == 0)
    def _():
        pltpu.sync_copy(cbuf_v, cnt_sh)

    @pl.loop(0, PCH, step=L)
    def _(i):
        ones_v[pl.ds(i, L)] = jnp.ones((L,), jnp.float32)

    plsc.subcore_barrier()

    nchunks = jnp.where(w == NW - 1, 1, NPW // PCH)

    @pl.loop(0, nchunks)
    def _(k):
        base = w * NPW + k * PCH
        pltpu.sync_copy(batch_hbm.at[pl.ds(base, PCH)], bidx_v)
        pltpu.sync_copy(h_hbm.at[pl.ds(base, PCH)], rows_v)
        pltpu.sync_copy(rows_v, pool_sh.at[bidx_v], add=True)
        pltpu.sync_copy(ones_v, cnt_sh.at[bidx_v], add=True)

    plsc.subcore_barrier()

    @pl.when(s < G // GPT)
    def _():
        pltpu.sync_copy(pool_sh.at[pl.ds(s * GPT, GPT)], pbuf_v)
        pltpu.sync_copy(pbuf_v, pool_hbm.at[c, pl.ds(s * GPT, GPT)])

    @pl.when(s == 0)
    def _():
        pltpu.sync_copy(cnt_sh, cbuf_v)
        pltpu.sync_copy(cbuf_v, cnt_hbm.at[pl.ds(c * CPAD, CPAD)])


def _dinv_body(dp_ref, o_ref):
    deg = jnp.sum(dp_ref[...], axis=0, keepdims=True) + 1.0
    o_ref[...] = lax.rsqrt(deg)


def _mm_pre_body(x_ref, w_ref, dinv_ref, o_ref):
    h = jnp.dot(x_ref[...], w_ref[...], preferred_element_type=jnp.float32)
    o_ref[...] = dinv_ref[...] * h


def _combine_relu_body(s_ref, g_ref, dinv_ref, b_ref, o_ref):
    v = dinv_ref[...] * (s_ref[0] + s_ref[1] + g_ref[...]) + b_ref[...]
    o_ref[...] = jnp.maximum(v, 0.0)


def _combine_body(s_ref, g_ref, dinv_ref, b_ref, o_ref):
    o_ref[...] = dinv_ref[...] * (s_ref[0] + s_ref[1] + g_ref[...]) + b_ref[...]


def _final_body(pp_ref, cc_ref, wl_ref, bl_ref, o_ref):
    def body(i, carry):
        p, c2 = carry
        return p + pp_ref[i], c2 + cc_ref[i]

    p0 = jnp.zeros((G, D), jnp.float32)
    c0 = jnp.zeros((G, 1), jnp.float32)
    p, c2 = lax.fori_loop(0, NC, body, (p0, c0))
    pooled = p / jnp.maximum(c2, 1.0)
    o_ref[...] = (
        jnp.dot(pooled, wl_ref[...], preferred_element_type=jnp.float32)
        + bl_ref[...]
    )


BR = 1000


def _mm_pre(x, w, dinv):
    return pl.pallas_call(
        _mm_pre_body,
        grid=(N // BR,),
        in_specs=[
            pl.BlockSpec((BR, D), lambda i: (i, 0)),
            pl.BlockSpec((D, D), lambda i: (0, 0)),
            pl.BlockSpec((BR, 1), lambda i: (i, 0)),
        ],
        out_specs=pl.BlockSpec((BR, D), lambda i: (i, 0)),
        out_shape=jax.ShapeDtypeStruct((N, D), jnp.float32),
    )(x, w, dinv)


def _combine(srt, g, dinv, b, relu):
    body = _combine_relu_body if relu else _combine_body
    return pl.pallas_call(
        body,
        grid=(N // BR,),
        in_specs=[
            pl.BlockSpec((NC, BR, D), lambda i: (0, i, 0)),
            pl.BlockSpec((BR, D), lambda i: (i, 0)),
            pl.BlockSpec((BR, 1), lambda i: (i, 0)),
            pl.BlockSpec((1, D), lambda i: (0, 0)),
        ],
        out_specs=pl.BlockSpec((BR, D), lambda i: (i, 0)),
        out_shape=jax.ShapeDtypeStruct((N, D), jnp.float32),
    )(srt, g, dinv, b)


def kernel(x, edge_index, batch, W1, b1, W2, b2, W3, b3, Wl, bl):
    src = edge_index[0].reshape(NW, NG, KJG, CH)
    dst = edge_index[1].reshape(NW, NG, KJG, CH)

    deg_part = _deg_kernel(dst.reshape(NW, KJ, CH)).reshape(NC, NPAD)
    dinv_flat = pl.pallas_call(
        _dinv_body,
        out_shape=jax.ShapeDtypeStruct((1, NPAD), jnp.float32),
    )(deg_part)
    dinv = dinv_flat.reshape(NPAD)[:N].reshape(N, 1)

    h = x
    for w, b, relu in ((W1, b1, True), (W2, b2, True), (W3, b3, False)):
        g = _mm_pre(h, w, dinv)
        srt = _edge_kernel(g, src, dst)
        h = _combine(srt, g, dinv, b.reshape(1, D), relu)

    pool_part, cnt_part = _pool_kernel(h, batch)
    cnt3 = cnt_part.reshape(NC, CPAD)[:, :G].reshape(NC, G, 1)
    y = pl.pallas_call(
        _final_body,
        out_shape=jax.ShapeDtypeStruct((G, 1), jnp.float32),
    )(pool_part, cnt3, Wl, bl.reshape(1, 1))
    return y


# speedup vs baseline: 2.6210x; 1.0762x over previous
import functools

import jax
import jax.numpy as jnp
from jax import lax
from jax.experimental import pallas as pl
from jax.experimental.pallas import tpu as pltpu
from jax.experimental.pallas import tpu_sc as plsc

N = 10000
E = 320000
D = 128
G = 64

NC = 2
NS = 16
NW = NC * NS
L = 16

CH = 80
EPW = E // NW
KJ = EPW // CH
NACC = N
ORT = 624
OCH = 48
OTAIL = N - NS * ORT
NG = 5
KJG = KJ // NG

NPAD = 10240
NPW = 320
PCH = 80

_mesh = plsc.VectorSubcoreMesh(core_axis_name="c", subcore_axis_name="s")


def _zero_1d(ref, n):
    @pl.loop(0, n, step=L)
    def _(i):
        ref[pl.ds(i, L)] = jnp.zeros((L,), jnp.float32)


def _fill_rows(ref, rows, val):
    @pl.loop(0, rows)
    def _(r):
        @pl.loop(0, ref.shape[1], step=L)
        def _(cc):
            ref[r, pl.ds(cc, L)] = jnp.full((L,), val, jnp.float32)


NPT = NPAD // NS


@functools.partial(
    pl.kernel,
    out_type=jax.ShapeDtypeStruct((NC * NPAD,), jnp.float32),
    mesh=_mesh,
    scratch_types=[
        pltpu.VMEM((KJ, CH), jnp.int32),
        pltpu.VMEM((CH,), jnp.float32),
        pltpu.VMEM((NPT,), jnp.float32),
        pltpu.VMEM_SHARED((NPAD,), jnp.float32),
    ],
)
def _deg_kernel(dst_hbm, out_hbm, idx_v, ones_v, buf_v, deg_sh):
    c = lax.axis_index("c")
    s = lax.axis_index("s")
    w = c * NS + s
    _zero_1d(buf_v, NPT)
    pltpu.sync_copy(buf_v, deg_sh.at[pl.ds(s * NPT, NPT)])

    @pl.loop(0, CH, step=L)
    def _(i):
        ones_v[pl.ds(i, L)] = jnp.ones((L,), jnp.float32)

    pltpu.sync_copy(dst_hbm.at[w], idx_v)
    plsc.subcore_barrier()

    @pl.loop(0, KJ)
    def _(j):
        pltpu.sync_copy(ones_v, deg_sh.at[idx_v.at[j]], add=True)

    plsc.subcore_barrier()
    pltpu.sync_copy(deg_sh.at[pl.ds(s * NPT, NPT)], buf_v)
    pltpu.sync_copy(buf_v, out_hbm.at[pl.ds(c * NPAD + s * NPT, NPT)])


@functools.partial(
    pl.kernel,
    out_type=jax.ShapeDtypeStruct((NC, N, D), jnp.float32),
    mesh=_mesh,
    scratch_types=[
        pltpu.VMEM((KJG, CH), jnp.int32),
        pltpu.VMEM((KJG, CH), jnp.int32),
        pltpu.VMEM((KJG, CH), jnp.int32),
        pltpu.VMEM((KJG, CH), jnp.int32),
        pltpu.VMEM((CH, D), jnp.float32),
        pltpu.VMEM((CH, D), jnp.float32),
        pltpu.VMEM((OCH, D), jnp.float32),
        pltpu.VMEM((OCH, D), jnp.float32),
        pltpu.VMEM_SHARED((NACC, D), jnp.float32),
        pltpu.SemaphoreType.DMA,
        pltpu.SemaphoreType.DMA,
        pltpu.SemaphoreType.DMA,
        pltpu.SemaphoreType.DMA,
    ],
)
def _edge_kernel(g_hbm, src_hbm, dst_hbm, out_hbm, srcA_v, dstA_v, srcB_v,
                 dstB_v, r0_v, r1_v, bufA_v, bufB_v, acc_sh, sem0, sem1,
                 semA, semB):
    c = lax.axis_index("c")
    s = lax.axis_index("s")
    w = c * NS + s
    _fill_rows(bufA_v, OCH, 0.0)

    @pl.loop(0, ORT // OCH)
    def _(z):
        pltpu.async_copy(bufA_v, acc_sh.at[pl.ds(s * ORT + z * OCH, OCH)],
                         semA)

    @pl.loop(0, ORT // OCH)
    def _(z):
        pltpu.make_async_copy(bufA_v, acc_sh.at[pl.ds(s * ORT, OCH)],
                              semA).wait()

    @pl.when(s == NS - 1)
    def _():
        pltpu.sync_copy(bufA_v.at[pl.ds(0, OTAIL)],
                        acc_sh.at[pl.ds(NS * ORT, OTAIL)])

    plsc.subcore_barrier()

    def _group(src_v, dst_v):
        def _wait_sc(buf, sem):
            pltpu.make_async_copy(buf, acc_sh.at[dst_v.at[0]], sem).wait()

        pltpu.sync_copy(g_hbm.at[src_v.at[0]], r0_v)
        pltpu.async_copy(r0_v, acc_sh.at[dst_v.at[0]], sem0, add=True)
        pltpu.sync_copy(g_hbm.at[src_v.at[1]], r1_v)
        pltpu.async_copy(r1_v, acc_sh.at[dst_v.at[1]], sem1, add=True)

        @pl.loop(1, (KJG - 1) // 2)
        def _(k):
            j = 2 * k
            _wait_sc(r0_v, sem0)
            pltpu.sync_copy(g_hbm.at[src_v.at[j]], r0_v)
            pltpu.async_copy(r0_v, acc_sh.at[dst_v.at[j]], sem0, add=True)
            _wait_sc(r1_v, sem1)
            pltpu.sync_copy(g_hbm.at[src_v.at[j + 1]], r1_v)
            pltpu.async_copy(r1_v, acc_sh.at[dst_v.at[j + 1]], sem1, add=True)

        _wait_sc(r0_v, sem0)
        pltpu.sync_copy(g_hbm.at[src_v.at[KJG - 1]], r0_v)
        pltpu.async_copy(r0_v, acc_sh.at[dst_v.at[KJG - 1]], sem0, add=True)
        _wait_sc(r0_v, sem0)
        _wait_sc(r1_v, sem1)

    pltpu.sync_copy(src_hbm.at[w, 0], srcA_v)
    pltpu.sync_copy(dst_hbm.at[w, 0], dstA_v)
    bufs = ((srcA_v, dstA_v), (srcB_v, dstB_v))
    for gi in range(NG):
        cur, nxt = bufs[gi % 2], bufs[(gi + 1) % 2]
        if gi + 1 < NG:
            pltpu.async_copy(src_hbm.at[w, gi + 1], nxt[0], semA)
            pltpu.async_copy(dst_hbm.at[w, gi + 1], nxt[1], semB)
        _group(*cur)
        if gi + 1 < NG:
            pltpu.make_async_copy(src_hbm.at[w, 0], nxt[0], semA).wait()
            pltpu.make_async_copy(dst_hbm.at[w, 0], nxt[1], semB).wait()

    plsc.subcore_barrier()

    ZC = ORT // OCH

    def _row(z):
        return s * ORT + z * OCH

    pltpu.sync_copy(acc_sh.at[pl.ds(_row(0), OCH)], bufA_v)
    pltpu.async_copy(bufA_v, out_hbm.at[c, pl.ds(_row(0), OCH)], semA)

    @pl.loop(0, (ZC - 1) // 2)
    def _(k):
        zb = 2 * k + 1
        @pl.when(k > 0)
        def _():
            pltpu.make_async_copy(bufB_v, out_hbm.at[c, pl.ds(_row(1), OCH)],
                                  semB).wait()
        pltpu.sync_copy(acc_sh.at[pl.ds(_row(zb), OCH)], bufB_v)
        pltpu.async_copy(bufB_v, out_hbm.at[c, pl.ds(_row(zb), OCH)], semB)
        pltpu.make_async_copy(bufA_v, out_hbm.at[c, pl.ds(_row(0), OCH)],
                              semA).wait()
        pltpu.sync_copy(acc_sh.at[pl.ds(_row(zb + 1), OCH)], bufA_v)
        pltpu.async_copy(bufA_v, out_hbm.at[c, pl.ds(_row(zb + 1), OCH)], semA)

    pltpu.make_async_copy(bufA_v, out_hbm.at[c, pl.ds(_row(0), OCH)],
                          semA).wait()
    pltpu.make_async_copy(bufB_v, out_hbm.at[c, pl.ds(_row(1), OCH)],
                          semB).wait()

    @pl.when(s == NS - 1)
    def _():
        pltpu.sync_copy(acc_sh.at[pl.ds(NS * ORT, OTAIL)],
                        bufA_v.at[pl.ds(0, OTAIL)])
        pltpu.sync_copy(bufA_v.at[pl.ds(0, OTAIL)],
                        out_hbm.at[c, pl.ds(NS * ORT, OTAIL)])


GPT = 8
CPAD = 128


@functools.partial(
    pl.kernel,
    out_type=(
        jax.ShapeDtypeStruct((NC, G, D), jnp.float32),
        jax.ShapeDtypeStruct((NC * CPAD,), jnp.float32),
    ),
    mesh=_mesh,
    scratch_types=[
        pltpu.VMEM((PCH,), jnp.int32),
        pltpu.VMEM((PCH,), jnp.float32),
        pltpu.VMEM((PCH, D), jnp.float32),
        pltpu.VMEM((GPT, D), jnp.float32),
        pltpu.VMEM((CPAD,), jnp.float32),
        pltpu.VMEM_SHARED((G, D), jnp.float32),
        pltpu.VMEM_SHARED((CPAD,), jnp.float32),
    ],
)
def _pool_kernel(h_hbm, batch_hbm, pool_hbm, cnt_hbm, bidx_v, ones_v, rows_v,
                 pbuf_v, cbuf_v, pool_sh, cnt_sh):
    c = lax.axis_index("c")
    s = lax.axis_index("s")
    w = c * NS + s
    _fill_rows(pbuf_v, GPT, 0.0)
    _zero_1d(cbuf_v, CPAD)

    @pl.when(s < G // GPT)
    def _():
        pltpu.sync_copy(pbuf_v, pool_sh.at[pl.ds(s * GPT, GPT)])

    @pl.when(s == 0)
    def _():
        pltpu.sync_copy(cbuf_v, cnt_sh)

    @pl.loop(0, PCH, step=L)
    def _(i):
        ones_v[pl.ds(i, L)] = jnp.ones((L,), jnp.float32)

    plsc.subcore_barrier()

    nchunks = jnp.where(w == NW - 1, 1, NPW // PCH)

    @pl.loop(0, nchunks)
    def _(k):
        base = w * NPW + k * PCH
        pltpu.sync_copy(batch_hbm.at[pl.ds(base, PCH)], bidx_v)
        pltpu.sync_copy(h_hbm.at[pl.ds(base, PCH)], rows_v)
        pltpu.sync_copy(rows_v, pool_sh.at[bidx_v], add=True)
        pltpu.sync_copy(ones_v, cnt_sh.at[bidx_v], add=True)

    plsc.subcore_barrier()

    @pl.when(s < G // GPT)
    def _():
        pltpu.sync_copy(pool_sh.at[pl.ds(s * GPT, GPT)], pbuf_v)
        pltpu.sync_copy(pbuf_v, pool_hbm.at[c, pl.ds(s * GPT, GPT)])

    @pl.when(s == 0)
    def _():
        pltpu.sync_copy(cnt_sh, cbuf_v)
        pltpu.sync_copy(cbuf_v, cnt_hbm.at[pl.ds(c * CPAD, CPAD)])


def _dinv_body(dp_ref, o_ref):
    deg = jnp.sum(dp_ref[...], axis=0, keepdims=True) + 1.0
    o_ref[...] = lax.rsqrt(deg)


def _mm_plain_body(x_ref, w_ref, o_ref):
    o_ref[...] = jnp.dot(x_ref[...], w_ref[...],
                         preferred_element_type=jnp.float32)


def _prescale_body(h_ref, dinv_ref, o_ref):
    o_ref[...] = dinv_ref[...] * h_ref[...]


def _fused_body(s_ref, g_ref, dinv_ref, b_ref, w_ref, o_ref):
    v = dinv_ref[...] * (s_ref[0] + s_ref[1] + g_ref[...]) + b_ref[...]
    o = jnp.maximum(v, 0.0)
    o_ref[...] = dinv_ref[...] * jnp.dot(
        o, w_ref[...], preferred_element_type=jnp.float32)


def _combine_body(s_ref, g_ref, dinv_ref, b_ref, o_ref):
    o_ref[...] = dinv_ref[...] * (s_ref[0] + s_ref[1] + g_ref[...]) + b_ref[...]


def _final_body(pp_ref, cc_ref, wl_ref, bl_ref, o_ref):
    def body(i, carry):
        p, c2 = carry
        return p + pp_ref[i], c2 + cc_ref[i]

    p0 = jnp.zeros((G, D), jnp.float32)
    c0 = jnp.zeros((G, 1), jnp.float32)
    p, c2 = lax.fori_loop(0, NC, body, (p0, c0))
    pooled = p / jnp.maximum(c2, 1.0)
    o_ref[...] = (
        jnp.dot(pooled, wl_ref[...], preferred_element_type=jnp.float32)
        + bl_ref[...]
    )


BR = 1000


def _mm_plain(x, w):
    return pl.pallas_call(
        _mm_plain_body,
        grid=(N // BR,),
        in_specs=[
            pl.BlockSpec((BR, D), lambda i: (i, 0)),
            pl.BlockSpec((D, D), lambda i: (0, 0)),
        ],
        out_specs=pl.BlockSpec((BR, D), lambda i: (i, 0)),
        out_shape=jax.ShapeDtypeStruct((N, D), jnp.float32),
    )(x, w)


def _prescale(h, dinv):
    return pl.pallas_call(
        _prescale_body,
        grid=(N // BR,),
        in_specs=[
            pl.BlockSpec((BR, D), lambda i: (i, 0)),
            pl.BlockSpec((BR, 1), lambda i: (i, 0)),
        ],
        out_specs=pl.BlockSpec((BR, D), lambda i: (i, 0)),
        out_shape=jax.ShapeDtypeStruct((N, D), jnp.float32),
    )(h, dinv)


def _fused(srt, g, dinv, b, w):
    return pl.pallas_call(
        _fused_body,
        grid=(N // BR,),
        in_specs=[
            pl.BlockSpec((NC, BR, D), lambda i: (0, i, 0)),
            pl.BlockSpec((BR, D), lambda i: (i, 0)),
            pl.BlockSpec((BR, 1), lambda i: (i, 0)),
            pl.BlockSpec((1, D), lambda i: (0, 0)),
            pl.BlockSpec((D, D), lambda i: (0, 0)),
        ],
        out_specs=pl.BlockSpec((BR, D), lambda i: (i, 0)),
        out_shape=jax.ShapeDtypeStruct((N, D), jnp.float32),
    )(srt, g, dinv, b, w)


def _combine(srt, g, dinv, b):
    return pl.pallas_call(
        _combine_body,
        grid=(N // BR,),
        in_specs=[
            pl.BlockSpec((NC, BR, D), lambda i: (0, i, 0)),
            pl.BlockSpec((BR, D), lambda i: (i, 0)),
            pl.BlockSpec((BR, 1), lambda i: (i, 0)),
            pl.BlockSpec((1, D), lambda i: (0, 0)),
        ],
        out_specs=pl.BlockSpec((BR, D), lambda i: (i, 0)),
        out_shape=jax.ShapeDtypeStruct((N, D), jnp.float32),
    )(srt, g, dinv, b)


def kernel(x, edge_index, batch, W1, b1, W2, b2, W3, b3, Wl, bl):
    src = edge_index[0].reshape(NW, NG, KJG, CH)
    dst = edge_index[1].reshape(NW, NG, KJG, CH)

    deg_part = _deg_kernel(dst.reshape(NW, KJ, CH)).reshape(NC, NPAD)
    dinv_flat = pl.pallas_call(
        _dinv_body,
        out_shape=jax.ShapeDtypeStruct((1, NPAD), jnp.float32),
    )(deg_part)
    dinv = dinv_flat.reshape(NPAD)[:N].reshape(N, 1)

    h1 = _mm_plain(x, W1)
    g = _prescale(h1, dinv)
    s1 = _edge_kernel(g, src, dst)
    g = _fused(s1, g, dinv, b1.reshape(1, D), W2)
    s2 = _edge_kernel(g, src, dst)
    g = _fused(s2, g, dinv, b2.reshape(1, D), W3)
    s3 = _edge_kernel(g, src, dst)
    h = _combine(s3, g, dinv, b3.reshape(1, D))

    pool_part, cnt_part = _pool_kernel(h, batch)
    cnt3 = cnt_part.reshape(NC, CPAD)[:, :G].reshape(NC, G, 1)
    y = pl.pallas_call(
        _final_body,
        out_shape=jax.ShapeDtypeStruct((G, 1), jnp.float32),
    )(pool_part, cnt3, Wl, bl.reshape(1, 1))
    return y
